# baseline (speedup 1.0000x reference)
import functools

import jax
import jax.numpy as jnp
from jax import lax
from jax.experimental import pallas as pl
from jax.experimental.pallas import tpu as pltpu
from jax.experimental.pallas import tpu_sc as plsc

_BATCH = 16384
_SEQ = 50
_DIM = 64
_NC = 2
_NS = 16
_NW = _NC * _NS
_CHUNK = 128
_NBLK = _SEQ * (_BATCH // _CHUNK)
_PER_W = _NBLK // _NW
_NB = 2

_mesh = plsc.VectorSubcoreMesh(core_axis_name="c", subcore_axis_name="s")


@functools.partial(
    pl.kernel,
    out_type=jax.ShapeDtypeStruct((_SEQ, 8, _BATCH // _CHUNK, 8, _CHUNK),
                                  jnp.float32),
    mesh=_mesh,
    compiler_params=pltpu.CompilerParams(use_tc_tiling_on_sc=False,
                                         needs_layout_passes=False,
                                         disable_bounds_checks=True),
    scratch_types=[
        pltpu.VMEM((_PER_W, _CHUNK), jnp.int32),
        pltpu.VMEM((_NB, _CHUNK, _DIM), jnp.float32),
        pltpu.VMEM((_NB, 8, 8, _CHUNK), jnp.float32),
        [pltpu.SemaphoreType.DMA] * _NB,
        [pltpu.SemaphoreType.DMA] * _NB,
    ],
)
def _embedding_gather(idx_hbm, table_hbm, out_hbm, idx_v, rows_v, t_v,
                      sg, ss):
    wid = lax.axis_index("s") * _NC + lax.axis_index("c")
    base_blk = wid * _PER_W

    pltpu.sync_copy(idx_hbm.at[pl.ds(base_blk, _PER_W)], idx_v)

    for b in range(_NB):
        pltpu.async_copy(table_hbm.at[idx_v.at[b]], rows_v.at[b], sg[b])

    lane = lax.iota(jnp.int32, 16)
    row_vecs = [lane + (g16 * 16) for g16 in range(8)]

    def body(grp, carry):
        for b in range(_NB):
            j = grp * _NB + b
            g = base_blk + j
            s = g >> 7
            b1 = g & 127

            pltpu.make_async_copy(
                table_hbm.at[idx_v.at[b]], rows_v.at[b], sg[b]).wait()

            @pl.when(j >= _NB)
            def _():
                pltpu.make_async_copy(
                    t_v.at[b], out_hbm.at[0, :, 0], ss[b]).wait()

            rows_b = rows_v.at[b]
            t_b = t_v.at[b]

            @plsc.parallel_loop(0, _DIM, step=1, unroll=8)
            def _tr(ee, rows_b=rows_b, t_b=t_b):
                col = jnp.broadcast_to(ee.astype(jnp.int32), (16,))
                for g16 in range(8):
                    vec = plsc.load_gather(rows_b, [row_vecs[g16], col])
                    t_b[ee >> 3, ee & 7, pl.ds(g16 * 16, 16)] = vec

            pltpu.async_copy(t_v.at[b], out_hbm.at[s, :, b1], ss[b])

            @pl.when(j + _NB < _PER_W)
            def _():
                pltpu.async_copy(
                    table_hbm.at[idx_v.at[j + _NB]], rows_v.at[b], sg[b])

        return carry

    lax.fori_loop(0, _PER_W // _NB, body, 0, unroll=False)

    for b in range(_NB):
        pltpu.make_async_copy(t_v.at[b], out_hbm.at[0, :, 0], ss[b]).wait()


def kernel(input_, weight):
    idx = jnp.transpose(input_).reshape(_NBLK, _CHUNK).astype(jnp.int32)
    out6 = _embedding_gather(idx, weight)
    return jnp.transpose(out6, (2, 4, 0, 1, 3)).reshape(_BATCH, _SEQ, _DIM)


# speedup vs baseline: 1.0059x; 1.0059x over previous
import functools

import jax
import jax.numpy as jnp
from jax import lax
from jax.experimental import pallas as pl
from jax.experimental.pallas import tpu as pltpu
from jax.experimental.pallas import tpu_sc as plsc

_BATCH = 16384
_SEQ = 50
_DIM = 64
_NC = 2
_NS = 16
_NW = _NC * _NS
_CHUNK = 128
_NBLK = _SEQ * (_BATCH // _CHUNK)
_PER_W = _NBLK // _NW
_NB = 4

_mesh = plsc.VectorSubcoreMesh(core_axis_name="c", subcore_axis_name="s")


@functools.partial(
    pl.kernel,
    out_type=jax.ShapeDtypeStruct((_SEQ, 8, _BATCH // _CHUNK, 8, _CHUNK),
                                  jnp.float32),
    mesh=_mesh,
    compiler_params=pltpu.CompilerParams(use_tc_tiling_on_sc=False,
                                         needs_layout_passes=False,
                                         disable_bounds_checks=True),
    scratch_types=[
        pltpu.VMEM((_PER_W, _CHUNK), jnp.int32),
        pltpu.VMEM((_NB, _CHUNK, _DIM), jnp.float32),
        pltpu.VMEM((_NB, 8, 8, _CHUNK), jnp.float32),
        [pltpu.SemaphoreType.DMA] * _NB,
        [pltpu.SemaphoreType.DMA] * _NB,
    ],
)
def _embedding_gather(idx_hbm, table_hbm, out_hbm, idx_v, rows_v, t_v,
                      sg, ss):
    wid = lax.axis_index("s") * _NC + lax.axis_index("c")
    base_blk = wid * _PER_W

    pltpu.sync_copy(idx_hbm.at[pl.ds(base_blk, _PER_W)], idx_v)

    for b in range(_NB):
        pltpu.async_copy(table_hbm.at[idx_v.at[b]], rows_v.at[b], sg[b])

    lane = lax.iota(jnp.int32, 16)
    row_vecs = [lane + (g16 * 16) for g16 in range(8)]

    def body(grp, carry):
        for b in range(_NB):
            j = grp * _NB + b
            g = base_blk + j
            s = g >> 7
            b1 = g & 127

            pltpu.make_async_copy(
                table_hbm.at[idx_v.at[b]], rows_v.at[b], sg[b]).wait()

            @pl.when(j >= _NB)
            def _():
                pltpu.make_async_copy(
                    t_v.at[b], out_hbm.at[0, :, 0], ss[b]).wait()

            rows_b = rows_v.at[b]
            t_b = t_v.at[b]

            @plsc.parallel_loop(0, _DIM, step=1, unroll=16)
            def _tr(ee, rows_b=rows_b, t_b=t_b):
                col = jnp.broadcast_to(ee.astype(jnp.int32), (16,))
                for g16 in range(8):
                    vec = plsc.load_gather(rows_b, [row_vecs[g16], col])
                    t_b[ee >> 3, ee & 7, pl.ds(g16 * 16, 16)] = vec

            pltpu.async_copy(t_v.at[b], out_hbm.at[s, :, b1], ss[b])

            @pl.when(j + _NB < _PER_W)
            def _():
                pltpu.async_copy(
                    table_hbm.at[idx_v.at[j + _NB]], rows_v.at[b], sg[b])

        return carry

    lax.fori_loop(0, _PER_W // _NB, body, 0, unroll=False)

    for b in range(_NB):
        pltpu.make_async_copy(t_v.at[b], out_hbm.at[0, :, 0], ss[b]).wait()


def kernel(input_, weight):
    idx = jnp.transpose(input_).reshape(_NBLK, _CHUNK).astype(jnp.int32)
    out6 = _embedding_gather(idx, weight)
    return jnp.transpose(out6, (2, 4, 0, 1, 3)).reshape(_BATCH, _SEQ, _DIM)
